# baseline (speedup 1.0000x reference)
import functools

import jax
import jax.numpy as jnp
from jax import lax
from jax.experimental import pallas as pl
from jax.experimental.pallas import tpu as pltpu
from jax.experimental.pallas import tpu_sc as plsc

N_NODES = 10000
D = 128
N_EDGES = 320000
NC = 2
NS = 16
N_PAD = 10240
HALF = N_PAD // 2
EDGES_PER_TILE = 20480
E_PAD = EDGES_PER_TILE * NS
STRIP = 512
N_STRIPS = EDGES_PER_TILE // STRIP
CHUNK = 32
ROWS_PER_TILE = N_PAD // NS
X_ROWS_PER_TILE = HALF // NS

_mesh = plsc.VectorSubcoreMesh(core_axis_name="c", subcore_axis_name="s")


@functools.partial(
    pl.kernel,
    mesh=_mesh,
    out_type=jax.ShapeDtypeStruct((NC, N_PAD, D), jnp.float32),
    compiler_params=pltpu.CompilerParams(needs_layout_passes=False),
    scratch_types=[
        pltpu.VMEM((STRIP,), jnp.int32),
        pltpu.VMEM((STRIP,), jnp.float32),
        pltpu.VMEM((STRIP + CHUNK,), jnp.int32),
        pltpu.VMEM((STRIP + CHUNK,), jnp.float32),
        pltpu.VMEM((CHUNK,), jnp.int32),
        pltpu.VMEM((CHUNK,), jnp.int32),
        pltpu.VMEM((CHUNK, D), jnp.float32),
        pltpu.VMEM_SHARED((HALF, D), jnp.float32),
        pltpu.VMEM_SHARED((N_PAD, D), jnp.float32),
        pltpu.SemaphoreType.DMA,
    ],
)
def _sc_aggregate(x_hbm, sd_hbm, w_hbm, out_hbm,
                  sdr_v, wr_v, sd_v, w_v, gidx, sidx, rows_f, x_sp,
                  acc_sh, sem):
    c = lax.axis_index("c")
    s = lax.axis_index("s")
    base = c * HALF
    NVR = D // 16

    zero16 = jnp.zeros((16,), jnp.float32)

    def _zrow(i, carry):
        for g in range(NVR):
            rows_f[i, pl.ds(g * 16, 16)] = zero16
        return carry

    lax.fori_loop(0, CHUNK, _zrow, 0)
    for k in range(ROWS_PER_TILE // CHUNK):
        pltpu.sync_copy(rows_f,
                        acc_sh.at[pl.ds(s * ROWS_PER_TILE + k * CHUNK, CHUNK)])
    pltpu.sync_copy(
        x_hbm.at[pl.ds(base + s * X_ROWS_PER_TILE, X_ROWS_PER_TILE)],
        x_sp.at[pl.ds(s * X_ROWS_PER_TILE, X_ROWS_PER_TILE)])
    plsc.subcore_barrier()

    def _strip(st, carry):
        pltpu.sync_copy(sd_hbm.at[s, pl.ds(st * STRIP, STRIP)], sdr_v)
        pltpu.sync_copy(w_hbm.at[s, pl.ds(st * STRIP, STRIP)], wr_v)

        lane = lax.iota(jnp.int32, 16)

        def _cgrp(g, cnt):
            sl = pl.ds(g * 16, 16)
            v = sdr_v[sl]
            wv = wr_v[sl]
            srcv = lax.bitwise_and(v, 0xFFFF)
            m = (srcv >= base) & (srcv < base + HALF)
            pos = plsc.cumsum(jnp.where(m, 1, 0))
            idx = cnt + jnp.where(m, pos - 1, 16 + lane)
            plsc.store_scatter(sd_v, [idx], v)
            plsc.store_scatter(w_v, [idx], wv)
            return cnt + pos[15]

        cnt = lax.fori_loop(0, STRIP // 16, _cgrp, jnp.int32(0))

        padv = jnp.zeros((16,), jnp.int32) + base
        sd_v[pl.ds(cnt, 16)] = padv
        sd_v[pl.ds(cnt + 16, 16)] = padv
        w_v[pl.ds(cnt, 16)] = zero16
        w_v[pl.ds(cnt + 16, 16)] = zero16
        nch = lax.div(cnt + (CHUNK - 1), jnp.int32(CHUNK))

        def _chunk(t, carry):
            for q in range(CHUNK // 16):
                sl = pl.ds(q * 16, 16)
                v = sd_v[pl.ds(t * CHUNK + q * 16, 16)]
                gidx[sl] = lax.bitwise_and(v, 0xFFFF) - base
                sidx[sl] = lax.shift_right_logical(v, 16)

            pltpu.async_copy(x_sp.at[gidx], rows_f, sem).wait()

            def _egrp(g, cc):
                wvec = w_v[pl.ds(t * CHUNK + g * 16, 16)]
                bb = g * 16
                for e in range(16):
                    w = wvec[e]
                    r = bb + e
                    for q in range(NVR):
                        sl = pl.ds(q * 16, 16)
                        rows_f[r, sl] = rows_f[r, sl] * w
                return cc

            lax.fori_loop(0, CHUNK // 16, _egrp, 0)

            pltpu.sync_copy(rows_f, acc_sh.at[sidx], add=True)
            return carry

        lax.fori_loop(0, nch, _chunk, 0)
        return carry

    lax.fori_loop(0, N_STRIPS, _strip, 0)

    plsc.subcore_barrier()
    pltpu.sync_copy(acc_sh.at[pl.ds(s * ROWS_PER_TILE, ROWS_PER_TILE)],
                    out_hbm.at[c, pl.ds(s * ROWS_PER_TILE, ROWS_PER_TILE)])


BLK = 1000


def _mlp_body(x_ref, p_ref, w1a_ref, w1b_ref, b1_ref, w2_ref, b2_ref, o_ref):
    agg = p_ref[0] + p_ref[1]
    h = jnp.dot(x_ref[...], w1a_ref[...], preferred_element_type=jnp.float32)
    h = h + jnp.dot(agg, w1b_ref[...], preferred_element_type=jnp.float32)
    h = h + b1_ref[...]
    h = jnp.maximum(h, 0.0)
    o_ref[...] = jnp.dot(h, w2_ref[...], preferred_element_type=jnp.float32) + b2_ref[...]


def _tc_mlp(x, partials, w1a, w1b, b1, w2, b2):
    return pl.pallas_call(
        _mlp_body,
        grid=(N_NODES // BLK,),
        in_specs=[
            pl.BlockSpec((BLK, D), lambda i: (i, 0)),
            pl.BlockSpec((NC, BLK, D), lambda i: (0, i, 0)),
            pl.BlockSpec((D, D), lambda i: (0, 0)),
            pl.BlockSpec((D, D), lambda i: (0, 0)),
            pl.BlockSpec((1, D), lambda i: (0, 0)),
            pl.BlockSpec((D, D), lambda i: (0, 0)),
            pl.BlockSpec((1, D), lambda i: (0, 0)),
        ],
        out_specs=pl.BlockSpec((BLK, D), lambda i: (i, 0)),
        out_shape=jax.ShapeDtypeStruct((N_NODES, D), jnp.float32),
    )(x, partials, w1a, w1b, b1, w2, b2)


def kernel(feature_data, edge_info, edge_weights, W_in, b_in, W_out, b_out):
    src = edge_info[0].astype(jnp.int32)
    dst = edge_info[1].astype(jnp.int32)
    w = edge_weights.astype(jnp.float32)
    pad = E_PAD - N_EDGES
    packed = jnp.concatenate(
        [lax.shift_left(dst, 16) | src, jnp.zeros((pad,), jnp.int32)]
    ).reshape(NS, EDGES_PER_TILE)
    w = jnp.concatenate([w, jnp.zeros((pad,), jnp.float32)]).reshape(
        NS, EDGES_PER_TILE)
    x_pad = jnp.zeros((N_PAD, D), jnp.float32).at[:N_NODES].set(feature_data)

    partials = _sc_aggregate(x_pad, packed, w)[:, :N_NODES]

    w1a = W_in[:, :D].T
    w1b = W_in[:, D:].T
    return _tc_mlp(feature_data, partials, w1a, w1b,
                   b_in.reshape(1, D), W_out.T, b_out.reshape(1, D))


# speedup vs baseline: 1.0194x; 1.0194x over previous
import functools

import jax
import jax.numpy as jnp
from jax import lax
from jax.experimental import pallas as pl
from jax.experimental.pallas import tpu as pltpu
from jax.experimental.pallas import tpu_sc as plsc

N_NODES = 10000
D = 128
N_EDGES = 320000
NC = 2
NS = 16
N_PAD = 10112
HALF = N_PAD // 2
EDGES_PER_TILE = 20480
E_PAD = EDGES_PER_TILE * NS
STRIP = 512
N_STRIPS = EDGES_PER_TILE // STRIP
CHUNK = 48
ROWS_PER_TILE = N_PAD // NS
X_ROWS_PER_TILE = HALF // 8

_mesh = plsc.VectorSubcoreMesh(core_axis_name="c", subcore_axis_name="s")


@functools.partial(
    pl.kernel,
    mesh=_mesh,
    out_type=jax.ShapeDtypeStruct((NC, N_PAD, D), jnp.float32),
    compiler_params=pltpu.CompilerParams(needs_layout_passes=False),
    scratch_types=[
        pltpu.VMEM((STRIP,), jnp.int32),
        pltpu.VMEM((STRIP,), jnp.float32),
        pltpu.VMEM((STRIP + CHUNK,), jnp.int32),
        pltpu.VMEM((STRIP + CHUNK,), jnp.float32),
        pltpu.VMEM((CHUNK,), jnp.int32),
        pltpu.VMEM((CHUNK,), jnp.int32),
        pltpu.VMEM((CHUNK, D), jnp.float32),
        pltpu.VMEM_SHARED((HALF, D), jnp.float32),
        pltpu.VMEM_SHARED((N_PAD, D), jnp.float32),
        pltpu.SemaphoreType.DMA,
    ],
)
def _sc_aggregate(x_hbm, sd_hbm, w_hbm, out_hbm,
                  sdr_v, wr_v, sd_v, w_v, gidx, sidx, rows_f, x_sp,
                  acc_sh, sem):
    c = lax.axis_index("c")
    s = lax.axis_index("s")
    base = c * HALF
    NVR = D // 16

    zero16 = jnp.zeros((16,), jnp.float32)

    def _zrow(i, carry):
        for g in range(NVR):
            rows_f[i, pl.ds(g * 16, 16)] = zero16
        return carry

    lax.fori_loop(0, CHUNK, _zrow, 0)
    for k in range(ROWS_PER_TILE // CHUNK):
        pltpu.sync_copy(rows_f,
                        acc_sh.at[pl.ds(s * ROWS_PER_TILE + k * CHUNK, CHUNK)])
    pltpu.sync_copy(
        rows_f.at[pl.ds(0, ROWS_PER_TILE % CHUNK)],
        acc_sh.at[pl.ds(s * ROWS_PER_TILE + (ROWS_PER_TILE // CHUNK) * CHUNK,
                        ROWS_PER_TILE % CHUNK)])

    @pl.when(s < 8)
    def _():
        pltpu.sync_copy(
            x_hbm.at[pl.ds(base + s * X_ROWS_PER_TILE, X_ROWS_PER_TILE)],
            x_sp.at[pl.ds(s * X_ROWS_PER_TILE, X_ROWS_PER_TILE)])

    plsc.subcore_barrier()

    def _strip(st, carry):
        pltpu.sync_copy(sd_hbm.at[s, pl.ds(st * STRIP, STRIP)], sdr_v)
        pltpu.sync_copy(w_hbm.at[s, pl.ds(st * STRIP, STRIP)], wr_v)

        lane = lax.iota(jnp.int32, 16)

        def _cgrp(g, cnt):
            sl = pl.ds(g * 16, 16)
            v = sdr_v[sl]
            wv = wr_v[sl]
            srcv = lax.bitwise_and(v, 0xFFFF)
            m = (srcv >= base) & (srcv < base + HALF)
            pos = plsc.cumsum(jnp.where(m, 1, 0))
            idx = cnt + jnp.where(m, pos - 1, 16 + lane)
            plsc.store_scatter(sd_v, [idx], v)
            plsc.store_scatter(w_v, [idx], wv)
            return cnt + pos[15]

        cnt = lax.fori_loop(0, STRIP // 16, _cgrp, jnp.int32(0))

        padv = jnp.zeros((16,), jnp.int32) + base
        for p in range(CHUNK // 16):
            sd_v[pl.ds(cnt + p * 16, 16)] = padv
            w_v[pl.ds(cnt + p * 16, 16)] = zero16
        nch = lax.div(cnt + (CHUNK - 1), jnp.int32(CHUNK))

        def _chunk(t, carry):
            for q in range(CHUNK // 16):
                sl = pl.ds(q * 16, 16)
                v = sd_v[pl.ds(t * CHUNK + q * 16, 16)]
                gidx[sl] = lax.bitwise_and(v, 0xFFFF) - base
                sidx[sl] = lax.shift_right_logical(v, 16)

            pltpu.async_copy(x_sp.at[gidx], rows_f, sem).wait()

            def _egrp(g, cc):
                wvec = w_v[pl.ds(t * CHUNK + g * 16, 16)]
                bb = g * 16
                for e in range(16):
                    w = wvec[e]
                    r = bb + e
                    for q in range(NVR):
                        sl = pl.ds(q * 16, 16)
                        rows_f[r, sl] = rows_f[r, sl] * w
                return cc

            lax.fori_loop(0, CHUNK // 16, _egrp, 0)

            pltpu.sync_copy(rows_f, acc_sh.at[sidx], add=True)
            return carry

        lax.fori_loop(0, nch, _chunk, 0)
        return carry

    lax.fori_loop(0, N_STRIPS, _strip, 0)

    plsc.subcore_barrier()
    pltpu.sync_copy(acc_sh.at[pl.ds(s * ROWS_PER_TILE, ROWS_PER_TILE)],
                    out_hbm.at[c, pl.ds(s * ROWS_PER_TILE, ROWS_PER_TILE)])


BLK = 1000


def _mlp_body(x_ref, p_ref, w1a_ref, w1b_ref, b1_ref, w2_ref, b2_ref, o_ref):
    agg = p_ref[0] + p_ref[1]
    h = jnp.dot(x_ref[...], w1a_ref[...], preferred_element_type=jnp.float32)
    h = h + jnp.dot(agg, w1b_ref[...], preferred_element_type=jnp.float32)
    h = h + b1_ref[...]
    h = jnp.maximum(h, 0.0)
    o_ref[...] = jnp.dot(h, w2_ref[...], preferred_element_type=jnp.float32) + b2_ref[...]


def _tc_mlp(x, partials, w1a, w1b, b1, w2, b2):
    return pl.pallas_call(
        _mlp_body,
        grid=(N_NODES // BLK,),
        in_specs=[
            pl.BlockSpec((BLK, D), lambda i: (i, 0)),
            pl.BlockSpec((NC, BLK, D), lambda i: (0, i, 0)),
            pl.BlockSpec((D, D), lambda i: (0, 0)),
            pl.BlockSpec((D, D), lambda i: (0, 0)),
            pl.BlockSpec((1, D), lambda i: (0, 0)),
            pl.BlockSpec((D, D), lambda i: (0, 0)),
            pl.BlockSpec((1, D), lambda i: (0, 0)),
        ],
        out_specs=pl.BlockSpec((BLK, D), lambda i: (i, 0)),
        out_shape=jax.ShapeDtypeStruct((N_NODES, D), jnp.float32),
    )(x, partials, w1a, w1b, b1, w2, b2)


def kernel(feature_data, edge_info, edge_weights, W_in, b_in, W_out, b_out):
    src = edge_info[0].astype(jnp.int32)
    dst = edge_info[1].astype(jnp.int32)
    w = edge_weights.astype(jnp.float32)
    pad = E_PAD - N_EDGES
    packed = jnp.concatenate(
        [lax.shift_left(dst, 16) | src, jnp.zeros((pad,), jnp.int32)]
    ).reshape(NS, EDGES_PER_TILE)
    w = jnp.concatenate([w, jnp.zeros((pad,), jnp.float32)]).reshape(
        NS, EDGES_PER_TILE)
    x_pad = jnp.zeros((N_PAD, D), jnp.float32).at[:N_NODES].set(feature_data)

    partials = _sc_aggregate(x_pad, packed, w)[:, :N_NODES]

    w1a = W_in[:, :D].T
    w1b = W_in[:, D:].T
    return _tc_mlp(feature_data, partials, w1a, w1b,
                   b_in.reshape(1, D), W_out.T, b_out.reshape(1, D))


# speedup vs baseline: 1.1445x; 1.1227x over previous
import functools

import jax
import jax.numpy as jnp
from jax import lax
from jax.experimental import pallas as pl
from jax.experimental.pallas import tpu as pltpu
from jax.experimental.pallas import tpu_sc as plsc

N_NODES = 10000
D = 128
N_EDGES = 320000
NC = 2
NS = 16
N_PAD = 10112
HALF = N_PAD // 2
EDGES_PER_TILE = 20480
E_PAD = EDGES_PER_TILE * NS
STRIP = 512
N_STRIPS = EDGES_PER_TILE // STRIP
CHUNK = 48
ROWS_PER_TILE = N_PAD // NS
X_ROWS_PER_TILE = HALF // 8

_mesh = plsc.VectorSubcoreMesh(core_axis_name="c", subcore_axis_name="s")


@functools.partial(
    pl.kernel,
    mesh=_mesh,
    out_type=jax.ShapeDtypeStruct((NC, N_PAD, D), jnp.float32),
    compiler_params=pltpu.CompilerParams(needs_layout_passes=False),
    scratch_types=[
        [pltpu.VMEM((STRIP,), jnp.int32) for _ in range(2)],
        [pltpu.VMEM((STRIP,), jnp.float32) for _ in range(2)],
        [pltpu.SemaphoreType.DMA for _ in range(2)],
        pltpu.VMEM((STRIP + CHUNK,), jnp.int32),
        pltpu.VMEM((STRIP + CHUNK,), jnp.float32),
        pltpu.VMEM((CHUNK,), jnp.int32),
        pltpu.VMEM((CHUNK,), jnp.int32),
        pltpu.VMEM((CHUNK, D), jnp.float32),
        pltpu.VMEM_SHARED((HALF, D), jnp.float32),
        pltpu.VMEM_SHARED((N_PAD, D), jnp.float32),
        pltpu.SemaphoreType.DMA,
    ],
)
def _sc_aggregate(x_hbm, sd_hbm, w_hbm, out_hbm,
                  sdr, wr, stsem, sd_v, w_v, gidx, sidx, rows_f, x_sp,
                  acc_sh, sem):
    c = lax.axis_index("c")
    s = lax.axis_index("s")
    base = c * HALF
    NVR = D // 16

    zero16 = jnp.zeros((16,), jnp.float32)

    def _zrow(i, carry):
        for g in range(NVR):
            rows_f[i, pl.ds(g * 16, 16)] = zero16
        return carry

    lax.fori_loop(0, CHUNK, _zrow, 0)
    for k in range(ROWS_PER_TILE // CHUNK):
        pltpu.sync_copy(rows_f,
                        acc_sh.at[pl.ds(s * ROWS_PER_TILE + k * CHUNK, CHUNK)])
    pltpu.sync_copy(
        rows_f.at[pl.ds(0, ROWS_PER_TILE % CHUNK)],
        acc_sh.at[pl.ds(s * ROWS_PER_TILE + (ROWS_PER_TILE // CHUNK) * CHUNK,
                        ROWS_PER_TILE % CHUNK)])

    @pl.when(s < 8)
    def _():
        pltpu.sync_copy(
            x_hbm.at[pl.ds(base + s * X_ROWS_PER_TILE, X_ROWS_PER_TILE)],
            x_sp.at[pl.ds(s * X_ROWS_PER_TILE, X_ROWS_PER_TILE)])

    plsc.subcore_barrier()

    def _stage(st, b):
        pltpu.async_copy(sd_hbm.at[s, pl.ds(st * STRIP, STRIP)], sdr[b],
                         stsem[b])
        pltpu.async_copy(w_hbm.at[s, pl.ds(st * STRIP, STRIP)], wr[b],
                         stsem[b])

    def _stage_wait(st, b):
        pltpu.make_async_copy(sd_hbm.at[s, pl.ds(st * STRIP, STRIP)], sdr[b],
                              stsem[b]).wait()
        pltpu.make_async_copy(w_hbm.at[s, pl.ds(st * STRIP, STRIP)], wr[b],
                              stsem[b]).wait()

    _stage(0, 0)

    def _strip(st, b, sdr_v, wr_v):
        _stage_wait(st, b)

        @pl.when(st + 1 < N_STRIPS)
        def _():
            _stage(st + 1, 1 - b)

        lane = lax.iota(jnp.int32, 16)

        def _cgrp(g, cnt):
            sl = pl.ds(g * 16, 16)
            v = sdr_v[sl]
            wv = wr_v[sl]
            srcv = lax.bitwise_and(v, 0xFFFF)
            m = (srcv >= base) & (srcv < base + HALF)
            pos = plsc.cumsum(jnp.where(m, 1, 0))
            idx = cnt + jnp.where(m, pos - 1, 16 + lane)
            plsc.store_scatter(sd_v, [idx], v)
            plsc.store_scatter(w_v, [idx], wv)
            return cnt + pos[15]

        cnt = lax.fori_loop(0, STRIP // 16, _cgrp, jnp.int32(0))

        padv = jnp.zeros((16,), jnp.int32) + base
        for p in range(CHUNK // 16):
            sd_v[pl.ds(cnt + p * 16, 16)] = padv
            w_v[pl.ds(cnt + p * 16, 16)] = zero16
        nch = lax.div(cnt + (CHUNK - 1), jnp.int32(CHUNK))

        def _chunk(t, carry):
            for q in range(CHUNK // 16):
                sl = pl.ds(q * 16, 16)
                v = sd_v[pl.ds(t * CHUNK + q * 16, 16)]
                gidx[sl] = lax.bitwise_and(v, 0xFFFF) - base
                sidx[sl] = lax.shift_right_logical(v, 16)

            pltpu.async_copy(x_sp.at[gidx], rows_f, sem).wait()

            def _egrp(g, cc):
                wvec = w_v[pl.ds(t * CHUNK + g * 16, 16)]
                bb = g * 16
                for e in range(16):
                    w = wvec[e]
                    r = bb + e
                    for q in range(NVR):
                        sl = pl.ds(q * 16, 16)
                        rows_f[r, sl] = rows_f[r, sl] * w
                return cc

            lax.fori_loop(0, CHUNK // 16, _egrp, 0)

            pltpu.sync_copy(rows_f, acc_sh.at[sidx], add=True)
            return carry

        lax.fori_loop(0, nch, _chunk, 0)

    def _pair(k, carry):
        for b in range(2):
            _strip(k * 2 + b, b, sdr[b], wr[b])
        return carry

    lax.fori_loop(0, N_STRIPS // 2, _pair, 0)

    plsc.subcore_barrier()
    pltpu.sync_copy(acc_sh.at[pl.ds(s * ROWS_PER_TILE, ROWS_PER_TILE)],
                    out_hbm.at[c, pl.ds(s * ROWS_PER_TILE, ROWS_PER_TILE)])


BLK = 1000


def _mlp_body(x_ref, p_ref, w1a_ref, w1b_ref, b1_ref, w2_ref, b2_ref, o_ref):
    agg = p_ref[0] + p_ref[1]
    h = jnp.dot(x_ref[...], w1a_ref[...], preferred_element_type=jnp.float32)
    h = h + jnp.dot(agg, w1b_ref[...], preferred_element_type=jnp.float32)
    h = h + b1_ref[...]
    h = jnp.maximum(h, 0.0)
    o_ref[...] = jnp.dot(h, w2_ref[...], preferred_element_type=jnp.float32) + b2_ref[...]


def _tc_mlp(x, partials, w1a, w1b, b1, w2, b2):
    return pl.pallas_call(
        _mlp_body,
        grid=(N_NODES // BLK,),
        in_specs=[
            pl.BlockSpec((BLK, D), lambda i: (i, 0)),
            pl.BlockSpec((NC, BLK, D), lambda i: (0, i, 0)),
            pl.BlockSpec((D, D), lambda i: (0, 0)),
            pl.BlockSpec((D, D), lambda i: (0, 0)),
            pl.BlockSpec((1, D), lambda i: (0, 0)),
            pl.BlockSpec((D, D), lambda i: (0, 0)),
            pl.BlockSpec((1, D), lambda i: (0, 0)),
        ],
        out_specs=pl.BlockSpec((BLK, D), lambda i: (i, 0)),
        out_shape=jax.ShapeDtypeStruct((N_NODES, D), jnp.float32),
    )(x, partials, w1a, w1b, b1, w2, b2)


def kernel(feature_data, edge_info, edge_weights, W_in, b_in, W_out, b_out):
    src = edge_info[0].astype(jnp.int32)
    dst = edge_info[1].astype(jnp.int32)
    w = edge_weights.astype(jnp.float32)
    pad = E_PAD - N_EDGES
    packed = jnp.concatenate(
        [lax.shift_left(dst, 16) | src, jnp.zeros((pad,), jnp.int32)]
    ).reshape(NS, EDGES_PER_TILE)
    w = jnp.concatenate([w, jnp.zeros((pad,), jnp.float32)]).reshape(
        NS, EDGES_PER_TILE)
    x_pad = jnp.zeros((N_PAD, D), jnp.float32).at[:N_NODES].set(feature_data)

    partials = _sc_aggregate(x_pad, packed, w)[:, :N_NODES]

    w1a = W_in[:, :D].T
    w1b = W_in[:, D:].T
    return _tc_mlp(feature_data, partials, w1a, w1b,
                   b_in.reshape(1, D), W_out.T, b_out.reshape(1, D))
